# Initial kernel scaffold; baseline (speedup 1.0000x reference)
import jax
import jax.numpy as jnp
from jax.experimental import pallas as pl


def kernel(node_feats, edge_index, graph_ids, W1, b1, W2, b2, gamma, beta, eps):
    raise NotImplementedError("write your pallas kernel here")


# speedup vs baseline: 4.4199x; 4.4199x over previous
import functools

import jax
import jax.numpy as jnp
from jax import lax
from jax.experimental import pallas as pl
from jax.experimental.pallas import tpu as pltpu
from jax.experimental.pallas import tpu_sc as plsc

G = 128
L = 5


@functools.lru_cache(maxsize=None)
def _make_edge_scatter(N, D, E):
  info = plsc.get_sparse_core_info()
  NC, NS = info.num_cores, info.num_subcores
  NW = NC * NS
  assert E % NW == 0
  epw = E // NW
  C = 80
  assert epw % C == 0
  nchunks = epw // C
  rpt = (N // NS) // 8 * 8
  extra = N - NS * rpt
  assert extra % 8 == 0
  mesh = plsc.VectorSubcoreMesh(core_axis_name="c", subcore_axis_name="s")

  @functools.partial(
      pl.kernel, mesh=mesh,
      out_type=jax.ShapeDtypeStruct((NC, N, D), jnp.float32),
      scratch_types=[
          pltpu.VMEM((C,), jnp.int32),
          pltpu.VMEM((C,), jnp.int32),
          pltpu.VMEM((C, D), jnp.float32),
          pltpu.VMEM_SHARED((N, D), jnp.float32),
          pltpu.SemaphoreType.DMA,
      ])
  def sc_kernel(h_hbm, src_hbm, dst_hbm, out_hbm, si, di, rows, acc, sem):
    c = lax.axis_index("c")
    s = lax.axis_index("s")
    wid = s * NC + c

    def zrow(i, carry):
      def zcol(j, carry2):
        rows[i, pl.ds(j * 16, 16)] = jnp.zeros((16,), jnp.float32)
        return carry2
      return lax.fori_loop(0, D // 16, zcol, carry)
    lax.fori_loop(0, C, zrow, 0)

    base_row = s * rpt
    nfull = rpt // C
    tail = rpt - nfull * C
    for k in range(nfull):
      pltpu.sync_copy(rows, acc.at[pl.ds(base_row + k * C, C)])
    if tail:
      pltpu.sync_copy(rows.at[pl.ds(0, tail)],
                      acc.at[pl.ds(base_row + nfull * C, tail)])
    if extra:
      @pl.when(s == NS - 1)
      def _zero_extra():
        pltpu.sync_copy(rows.at[pl.ds(0, extra)],
                        acc.at[pl.ds(NS * rpt, extra)])
    plsc.subcore_barrier()

    ebase = wid * epw
    def chunk(k, carry):
      off = ebase + k * C
      pltpu.sync_copy(src_hbm.at[pl.ds(off, C)], si)
      pltpu.sync_copy(dst_hbm.at[pl.ds(off, C)], di)
      pltpu.async_copy(h_hbm.at[si], rows, sem).wait()
      pltpu.sync_copy(rows, acc.at[di], add=True)
      return carry
    lax.fori_loop(0, nchunks, chunk, 0)

    plsc.subcore_barrier()
    pltpu.sync_copy(acc.at[pl.ds(base_row, rpt)],
                    out_hbm.at[c, pl.ds(base_row, rpt)])
    if extra:
      @pl.when(s == NS - 1)
      def _write_extra():
        pltpu.sync_copy(acc.at[pl.ds(NS * rpt, extra)],
                        out_hbm.at[c, pl.ds(NS * rpt, extra)])

  return sc_kernel


def _dot3(x, w):
  d = functools.partial(jnp.dot, preferred_element_type=jnp.float32)
  return d(x.astype(jnp.bfloat16), w.astype(jnp.bfloat16))


def _mlp_body(h_ref, na_ref, nb_ref, w1_ref, b1_ref, w2_ref, b2_ref,
              g_ref, be_ref, eps_ref):
  z = (1.0 + eps_ref[0, 0]) * h_ref[...] + na_ref[...] + nb_ref[...]
  y = _dot3(z, w1_ref[...]) + b1_ref[...]
  y = jnp.maximum(y, 0.0)
  y = _dot3(y, w2_ref[...]) + b2_ref[...]
  mu = jnp.mean(y, axis=0, keepdims=True)
  d = y - mu
  var = jnp.mean(d * d, axis=0, keepdims=True)
  v = var + 1e-5
  r = lax.rsqrt(v)
  r = r * (1.5 - 0.5 * v * r * r)
  return d * r * g_ref[...] + be_ref[...]


def _mlp_mid_kernel(h_ref, na_ref, nb_ref, w1_ref, b1_ref, w2_ref, b2_ref,
                    g_ref, be_ref, eps_ref, o_ref):
  out = _mlp_body(h_ref, na_ref, nb_ref, w1_ref, b1_ref, w2_ref, b2_ref,
                  g_ref, be_ref, eps_ref)
  o_ref[...] = jnp.maximum(out, 0.0)


def _mlp_last_kernel(h_ref, na_ref, nb_ref, w1_ref, b1_ref, w2_ref, b2_ref,
                     g_ref, be_ref, eps_ref, ids_ref, o_ref):
  out = _mlp_body(h_ref, na_ref, nb_ref, w1_ref, b1_ref, w2_ref, b2_ref,
                  g_ref, be_ref, eps_ref)
  n = out.shape[0]
  seg = lax.broadcasted_iota(jnp.int32, (G, n), 0)
  m = (seg == ids_ref[...]).astype(jnp.float32)
  sums = jnp.dot(m, out, preferred_element_type=jnp.float32,
                 precision=lax.Precision.HIGHEST)
  counts = jnp.sum(m, axis=1, keepdims=True)
  o_ref[...] = sums / jnp.maximum(counts, 1.0)


@functools.lru_cache(maxsize=None)
def _make_mlp(N, D, last):
  if last:
    return pl.pallas_call(
        _mlp_last_kernel,
        out_shape=jax.ShapeDtypeStruct((G, D), jnp.float32))
  return pl.pallas_call(
      _mlp_mid_kernel,
      out_shape=jax.ShapeDtypeStruct((N, D), jnp.float32))


def kernel(node_feats, edge_index, graph_ids, W1, b1, W2, b2, gamma, beta,
           eps):
  N, D = node_feats.shape
  E = edge_index.shape[1]
  src = edge_index[0]
  dst = edge_index[1]
  ids = graph_ids.reshape(1, N)

  edge_scatter = _make_edge_scatter(N, D, E)
  h = node_feats
  for l in range(L):
    partials = edge_scatter(h, src, dst)
    args = (h, partials[0], partials[1], W1[l], b1[l].reshape(1, 2 * D),
            W2[l], b2[l].reshape(1, D), gamma[l].reshape(1, D),
            beta[l].reshape(1, D), eps[l].reshape(1, 1))
    if l < L - 1:
      h = _make_mlp(N, D, False)(*args)
    else:
      h = _make_mlp(N, D, True)(*args, ids)
  return h
